# baseline (speedup 1.0000x reference)
import functools

import jax
import jax.numpy as jnp
from jax import lax
from jax.experimental import pallas as pl
from jax.experimental.pallas import tpu as pltpu
from jax.experimental.pallas import tpu_sc as plsc

_C = 150
_H = 512
_W = 512
_HW = _H * _W

_NC = 2
_NS = 16
_L = 16
_NW = _NC * _NS
_PPW = _HW // _NW
_CCH = 6
_NCHUNK = _C // _CCH
_BUF = _CCH * _PPW

_mesh = plsc.VectorSubcoreMesh(
    core_axis_name="c", subcore_axis_name="s",
    num_cores=_NC, num_subcores=_NS)


@functools.partial(
    pl.kernel,
    out_type=jax.ShapeDtypeStruct((_C * _HW,), jnp.float32),
    mesh=_mesh,
    compiler_params=pltpu.CompilerParams(needs_layout_passes=False),
    scratch_types=[
        pltpu.VMEM((_PPW,), jnp.int32),
        pltpu.VMEM((_BUF,), jnp.float32),
        pltpu.VMEM((_BUF,), jnp.float32),
        pltpu.SemaphoreType.DMA,
        pltpu.SemaphoreType.DMA,
    ],
)
def _sc_onehot(lab_hbm, out_hbm, g_v, buf_a, buf_b, sem_a, sem_b):
    wid = lax.axis_index("s") * _NC + lax.axis_index("c")
    base_p = wid * _PPW

    pltpu.sync_copy(lab_hbm.at[pl.ds(base_p, _PPW)], g_v)

    lane = lax.iota(jnp.int32, _L)

    def init_g(i, carry):
        sl = pl.ds(i * _L, _L)
        g_v[sl] = g_v[sl] * _PPW + (i * _L + lane)
        return carry

    lax.fori_loop(0, _PPW // _L, init_g, 0)

    ones = jnp.full((_L,), 1.0, jnp.float32)
    zeros16 = jnp.zeros((_L,), jnp.float32)

    bufs = (buf_a, buf_b)
    sems = (sem_a, sem_b)
    pending = [None, None]
    for k in range(_NCHUNK):
        b = k % 2
        buf = bufs[b]
        if pending[b] is not None:
            for hnd in pending[b]:
                hnd.wait()

        def clear(i, carry, buf=buf):
            buf[pl.ds(i * _L, _L)] = zeros16
            return carry

        lax.fori_loop(0, _BUF // _L, clear, 0)

        off = k * _BUF

        def scat(i, carry, buf=buf, off=off):
            idx = g_v[pl.ds(i * _L, _L)] - off
            msk = (idx >= 0) & (idx < _BUF)
            plsc.store_scatter(buf, [idx], ones, mask=msk)
            return carry

        lax.fori_loop(0, _PPW // _L, scat, 0)

        hnds = []
        for j in range(_CCH):
            c = k * _CCH + j
            hnds.append(pltpu.async_copy(
                buf.at[pl.ds(j * _PPW, _PPW)],
                out_hbm.at[pl.ds(c * _HW + base_p, _PPW)],
                sems[b]))
        pending[b] = hnds

    for b in range(2):
        for hnd in pending[b]:
            hnd.wait()


def kernel(label):
    flat = _sc_onehot(label.reshape(_HW))
    return flat.reshape(_C, _H, _W)


# speedup vs baseline: 1.7068x; 1.7068x over previous
import functools

import jax
import jax.numpy as jnp
from jax import lax
from jax.experimental import pallas as pl
from jax.experimental.pallas import tpu as pltpu
from jax.experimental.pallas import tpu_sc as plsc

_C = 150
_H = 512
_W = 512
_HW = _H * _W

_NC = 2
_NS = 16
_L = 16
_NW = _NC * _NS
_PPW = _HW // _NW
_CCH = 6
_NCHUNK = _C // _CCH
_BUF = _CCH * _PPW

_mesh = plsc.VectorSubcoreMesh(
    core_axis_name="c", subcore_axis_name="s",
    num_cores=_NC, num_subcores=_NS)


@functools.partial(
    pl.kernel,
    out_type=jax.ShapeDtypeStruct((_C * _HW,), jnp.float32),
    mesh=_mesh,
    compiler_params=pltpu.CompilerParams(needs_layout_passes=False),
    scratch_types=[
        pltpu.VMEM((_PPW,), jnp.int32),
        pltpu.VMEM((_BUF,), jnp.float32),
        pltpu.VMEM((_BUF,), jnp.float32),
        pltpu.SemaphoreType.DMA,
        pltpu.SemaphoreType.DMA,
    ],
)
def _sc_onehot(lab_hbm, out_hbm, g_v, buf_a, buf_b, sem_a, sem_b):
    wid = lax.axis_index("s") * _NC + lax.axis_index("c")
    base_p = wid * _PPW

    pltpu.sync_copy(lab_hbm.at[pl.ds(base_p, _PPW)], g_v)

    lane = lax.iota(jnp.int32, _L)
    _U = 8

    def init_g(i, carry):
        for u in range(_U):
            sl = pl.ds((i * _U + u) * _L, _L)
            g_v[sl] = g_v[sl] * _PPW + ((i * _U + u) * _L + lane)
        return carry

    lax.fori_loop(0, _PPW // _L // _U, init_g, 0)

    ones = jnp.full((_L,), 1.0, jnp.float32)
    zeros16 = jnp.zeros((_L,), jnp.float32)

    def full_clear(buf):
        def body(i, carry):
            for u in range(_U):
                buf[pl.ds((i * _U + u) * _L, _L)] = zeros16
            return carry
        lax.fori_loop(0, _BUF // _L // _U, body, 0)

    def scan_scatter(buf, off, val):
        def body(i, carry):
            for u in range(_U):
                idx = g_v[pl.ds((i * _U + u) * _L, _L)] - off
                msk = (idx >= 0) & (idx < _BUF)
                plsc.store_scatter(buf, [idx], val, mask=msk)
            return carry
        lax.fori_loop(0, _PPW // _L // _U, body, 0)

    bufs = (buf_a, buf_b)
    sems = (sem_a, sem_b)
    pending = [None, None]
    for k in range(_NCHUNK):
        b = k % 2
        buf = bufs[b]
        if pending[b] is not None:
            for hnd in pending[b]:
                hnd.wait()
            scan_scatter(buf, (k - 2) * _BUF, zeros16)
        else:
            full_clear(buf)

        scan_scatter(buf, k * _BUF, ones)

        hnds = []
        for j in range(_CCH):
            c = k * _CCH + j
            hnds.append(pltpu.async_copy(
                buf.at[pl.ds(j * _PPW, _PPW)],
                out_hbm.at[pl.ds(c * _HW + base_p, _PPW)],
                sems[b]))
        pending[b] = hnds

    for b in range(2):
        for hnd in pending[b]:
            hnd.wait()


def kernel(label):
    flat = _sc_onehot(label.reshape(_HW))
    return flat.reshape(_C, _H, _W)


# speedup vs baseline: 2.4581x; 1.4402x over previous
import functools

import jax
import jax.numpy as jnp
from jax import lax
from jax.experimental import pallas as pl
from jax.experimental.pallas import tpu as pltpu
from jax.experimental.pallas import tpu_sc as plsc

_C = 150
_H = 512
_W = 512
_HW = _H * _W

_NC = 2
_NS = 16
_L = 16
_NW = _NC * _NS
_RPW = _H // _NW
_PPW = _HW // _NW
_CCH = 6
_NCHUNK = _C // _CCH
_BUF = _CCH * _PPW

_mesh = plsc.VectorSubcoreMesh(
    core_axis_name="c", subcore_axis_name="s",
    num_cores=_NC, num_subcores=_NS)


@functools.partial(
    pl.kernel,
    out_type=jax.ShapeDtypeStruct((_C, _H, _W), jnp.float32),
    mesh=_mesh,
    compiler_params=pltpu.CompilerParams(needs_layout_passes=False),
    scratch_types=[
        pltpu.VMEM((_PPW,), jnp.int32),
        pltpu.VMEM((_CCH, _RPW, _W), jnp.float32),
        pltpu.VMEM((_CCH, _RPW, _W), jnp.float32),
        pltpu.SemaphoreType.DMA,
        pltpu.SemaphoreType.DMA,
    ],
)
def _sc_onehot(lab_hbm, out_hbm, g_v, buf_a, buf_b, sem_a, sem_b):
    wid = lax.axis_index("s") * _NC + lax.axis_index("c")
    row0 = wid * _RPW

    pltpu.sync_copy(lab_hbm.at[pl.ds(wid * _PPW, _PPW)], g_v)

    lane = lax.iota(jnp.int32, _L)
    _U = 8

    ones = jnp.full((_L,), 1.0, jnp.float32)
    zeros16 = jnp.zeros((_L,), jnp.float32)

    def scan_scatter(buf, c0, val):
        def body(i, carry):
            for u in range(_U):
                q0 = (i * _U + u) * _L
                cc = g_v[pl.ds(q0, _L)] - c0
                msk = (cc >= 0) & (cc < _CCH)
                qv = q0 + lane
                plsc.store_scatter(
                    buf,
                    [cc, jax.lax.shift_right_logical(qv, 9), qv & (_W - 1)],
                    val, mask=msk)
            return carry
        lax.fori_loop(0, _PPW // _L // _U, body, 0)

    def full_clear(buf):
        flat = _BUF
        def body(i, carry):
            for u in range(_U):
                q0 = (i * _U + u) * _L
                buf[q0 // (_RPW * _W), (q0 // _W) % _RPW,
                    pl.ds(q0 % _W, _L)] = zeros16
            return carry
        lax.fori_loop(0, flat // _L // _U, body, 0)

    bufs = (buf_a, buf_b)
    sems = (sem_a, sem_b)
    pending = [None, None]
    for k in range(_NCHUNK):
        b = k % 2
        buf = bufs[b]
        if pending[b] is not None:
            for hnd in pending[b]:
                hnd.wait()
            scan_scatter(buf, (k - 2) * _CCH, zeros16)
        else:
            full_clear(buf)

        scan_scatter(buf, k * _CCH, ones)

        hnds = []
        for j in range(_CCH):
            c = k * _CCH + j
            hnds.append(pltpu.async_copy(
                buf.at[j],
                out_hbm.at[c, pl.ds(row0, _RPW), :],
                sems[b]))
        pending[b] = hnds

    for b in range(2):
        for hnd in pending[b]:
            hnd.wait()


def kernel(label):
    return _sc_onehot(label.reshape(_HW))


# speedup vs baseline: 8.1628x; 3.3208x over previous
import functools

import jax
import jax.numpy as jnp
from jax import lax
from jax.experimental import pallas as pl
from jax.experimental.pallas import tpu as pltpu
from jax.experimental.pallas import tpu_sc as plsc

_C = 150
_H = 512
_W = 512
_HW = _H * _W

_NC = 2
_NS = 16
_L = 16
_NW = _NC * _NS
_RPW = _H // _NW
_PPW = _HW // _NW
_CCH = 7
_NCHUNK = -(-_C // _CCH)
_BROWS = _CCH * _RPW

_mesh = plsc.VectorSubcoreMesh(
    core_axis_name="c", subcore_axis_name="s",
    num_cores=_NC, num_subcores=_NS)


@functools.partial(
    pl.kernel,
    out_type=jax.ShapeDtypeStruct((_C, _H, _W), jnp.float32),
    mesh=_mesh,
    compiler_params=pltpu.CompilerParams(needs_layout_passes=False),
    scratch_types=[
        pltpu.VMEM((_PPW,), jnp.int32),
        pltpu.VMEM((_BROWS, _W), jnp.float32),
        pltpu.VMEM((_BROWS, _W), jnp.float32),
        pltpu.SemaphoreType.DMA,
        pltpu.SemaphoreType.DMA,
    ],
)
def _sc_onehot(lab_hbm, out_hbm, r_v, buf_a, buf_b, sem_a, sem_b):
    wid = lax.axis_index("s") * _NC + lax.axis_index("c")
    row0 = wid * _RPW

    pltpu.sync_copy(lab_hbm.at[pl.ds(wid * _PPW, _PPW)], r_v)

    lane = lax.iota(jnp.int32, _L)
    ones = jnp.full((_L,), 1.0, jnp.float32)
    zeros16 = jnp.zeros((_L,), jnp.float32)
    bound = jnp.uint32(_BROWS)

    @plsc.parallel_loop(0, _PPW, step=_L, unroll=8)
    def _init_r(q0):
        sl = pl.ds(q0, _L)
        r_v[sl] = r_v[sl] * _RPW + jax.lax.shift_right_logical(q0, 9)

    def full_clear(buf):
        @plsc.parallel_loop(0, _BROWS * _W, step=_L, unroll=8)
        def _clr(q0):
            buf[jax.lax.shift_right_logical(q0, 9),
                pl.ds(q0 & (_W - 1), _L)] = zeros16

    def scan_chunk(buf, roff_new, roff_old):
        @plsc.parallel_loop(0, _PPW, step=_L, unroll=8)
        def _scan(q0):
            rv = r_v[pl.ds(q0, _L)]
            colv = (q0 & (_W - 1)) + lane
            row_new = rv - roff_new
            msk_new = plsc.bitcast(row_new, jnp.uint32) < bound
            plsc.store_scatter(buf, [row_new, colv], ones, mask=msk_new)
            if roff_old is not None:
                row_old = rv - roff_old
                msk_old = plsc.bitcast(row_old, jnp.uint32) < bound
                plsc.store_scatter(buf, [row_old, colv], zeros16, mask=msk_old)

    bufs = (buf_a, buf_b)
    sems = (sem_a, sem_b)
    pending = [None, None]
    for k in range(_NCHUNK):
        b = k % 2
        buf = bufs[b]
        if pending[b] is not None:
            for hnd in pending[b]:
                hnd.wait()
            scan_chunk(buf, k * _BROWS, (k - 2) * _BROWS)
        else:
            full_clear(buf)
            scan_chunk(buf, k * _BROWS, None)

        hnds = []
        for j in range(_CCH):
            c = k * _CCH + j
            if c >= _C:
                break
            hnds.append(pltpu.async_copy(
                buf.at[pl.ds(j * _RPW, _RPW), :],
                out_hbm.at[c, pl.ds(row0, _RPW), :],
                sems[b]))
        pending[b] = hnds

    for b in range(2):
        for hnd in pending[b]:
            hnd.wait()


def kernel(label):
    return _sc_onehot(label.reshape(_HW))


# speedup vs baseline: 8.2247x; 1.0076x over previous
import functools

import jax
import jax.numpy as jnp
from jax import lax
from jax.experimental import pallas as pl
from jax.experimental.pallas import tpu as pltpu
from jax.experimental.pallas import tpu_sc as plsc

_C = 150
_H = 512
_W = 512
_HW = _H * _W

_NC = 2
_NS = 16
_L = 16
_NW = _NC * _NS
_RPW = _H // _NW
_PPW = _HW // _NW
_CCH = 7
_NCHUNK = -(-_C // _CCH)
_BROWS = _CCH * _RPW

_mesh = plsc.VectorSubcoreMesh(
    core_axis_name="c", subcore_axis_name="s",
    num_cores=_NC, num_subcores=_NS)


@functools.partial(
    pl.kernel,
    out_type=jax.ShapeDtypeStruct((_C, _H, _W), jnp.float32),
    mesh=_mesh,
    compiler_params=pltpu.CompilerParams(needs_layout_passes=False),
    scratch_types=[
        pltpu.VMEM((_RPW, _W), jnp.int32),
        pltpu.VMEM((_BROWS, _W), jnp.float32),
        pltpu.VMEM((_BROWS, _W), jnp.float32),
        pltpu.SemaphoreType.DMA,
        pltpu.SemaphoreType.DMA,
    ],
)
def _sc_onehot(lab_hbm, out_hbm, r_v, buf_a, buf_b, sem_a, sem_b):
    wid = lax.axis_index("s") * _NC + lax.axis_index("c")
    row0 = wid * _RPW

    pltpu.sync_copy(lab_hbm.at[pl.ds(row0, _RPW), :], r_v)

    lane = lax.iota(jnp.int32, _L)
    ones = jnp.full((_L,), 1.0, jnp.float32)
    zeros16 = jnp.zeros((_L,), jnp.float32)
    bound = jnp.uint32(_BROWS)

    @plsc.parallel_loop(0, _PPW, step=_L, unroll=8)
    def _init_r(q0):
        h = jax.lax.shift_right_logical(q0, 9)
        sl = pl.ds(q0 & (_W - 1), _L)
        r_v[h, sl] = r_v[h, sl] * _RPW + h

    def full_clear(buf):
        @plsc.parallel_loop(0, _BROWS * _W, step=_L, unroll=8)
        def _clr(q0):
            buf[jax.lax.shift_right_logical(q0, 9),
                pl.ds(q0 & (_W - 1), _L)] = zeros16

    def scan_chunk(buf, roff_new, roff_old):
        @plsc.parallel_loop(0, _PPW, step=_L, unroll=8)
        def _scan(q0):
            rv = r_v[jax.lax.shift_right_logical(q0, 9),
                     pl.ds(q0 & (_W - 1), _L)]
            colv = (q0 & (_W - 1)) + lane
            row_new = rv - roff_new
            msk_new = plsc.bitcast(row_new, jnp.uint32) < bound
            plsc.store_scatter(buf, [row_new, colv], ones, mask=msk_new)
            if roff_old is not None:
                row_old = rv - roff_old
                msk_old = plsc.bitcast(row_old, jnp.uint32) < bound
                plsc.store_scatter(buf, [row_old, colv], zeros16, mask=msk_old)

    bufs = (buf_a, buf_b)
    sems = (sem_a, sem_b)
    pending = [None, None]
    for k in range(_NCHUNK):
        b = k % 2
        buf = bufs[b]
        if pending[b] is not None:
            for hnd in pending[b]:
                hnd.wait()
            scan_chunk(buf, k * _BROWS, (k - 2) * _BROWS)
        else:
            full_clear(buf)
            scan_chunk(buf, k * _BROWS, None)

        hnds = []
        for j in range(_CCH):
            c = k * _CCH + j
            if c >= _C:
                break
            hnds.append(pltpu.async_copy(
                buf.at[pl.ds(j * _RPW, _RPW), :],
                out_hbm.at[c, pl.ds(row0, _RPW), :],
                sems[b]))
        pending[b] = hnds

    for b in range(2):
        for hnd in pending[b]:
            hnd.wait()


def kernel(label):
    return _sc_onehot(label)
